# baseline (speedup 1.0000x reference)
import functools

import jax
import jax.numpy as jnp
from jax import lax
from jax.experimental import pallas as pl
from jax.experimental.pallas import tpu as pltpu
from jax.experimental.pallas import tpu_sc as plsc

NC = 2
NS = 16
L = 16
NW = NC * NS

F32 = jnp.float32


def _edge_sum_body(ea_ref, g_ref, s_ref, ssq_ref):
    i = pl.program_id(0)
    sb = jnp.dot(ea_ref[...], g_ref[...], preferred_element_type=F32)
    s_ref[...] = sb

    @pl.when(i == 0)
    def _():
        ssq_ref[...] = jnp.zeros_like(ssq_ref)

    ssq_ref[...] += jnp.sum(sb * sb).reshape(1, 1)


def _mlp_body(x_ref, wp_ref, bp_ref, w1_ref, o_ref):
    h = jnp.dot(x_ref[...], wp_ref[...], preferred_element_type=F32)
    h = jnp.maximum(h + bp_ref[...], 0.0)
    o_ref[...] = jnp.dot(h, w1_ref[...], preferred_element_type=F32)


def _ep_body(flag_ref, a_ref, u_ref, v_ref, hw_ref, b_ref, w_ref, bout_ref,
             o_ref):
    asum = a_ref[...]
    h = asum * u_ref[...] + hw_ref[...] * v_ref[...] + b_ref[...]
    f = flag_ref[0, 0]
    h = f * jnp.maximum(h, 0.0) + (1.0 - f) * h
    o_ref[...] = (jnp.dot(h, w_ref[...], preferred_element_type=F32)
                  + bout_ref[...])


def _fast_rsqrt(x):
    i = plsc.bitcast(x, jnp.int32)
    i = jnp.int32(0x5F3759DF) - lax.shift_right_arithmetic(i, 1)
    y = plsc.bitcast(i, F32)
    half = x * 0.5
    for _ in range(4):
        y = y * (1.5 - half * y * y)
    return y


def _make_deg_kernel(NP, EROWS_T):
    npv = NP // NS
    nv = npv // L
    NPH = NP // 2
    HR = EROWS_T // 2
    CAP = HR * 128
    EPAD = NS * EROWS_T * 128
    mesh = plsc.VectorSubcoreMesh(
        core_axis_name="c", subcore_axis_name="s",
        num_cores=1, num_subcores=NS)

    @functools.partial(
        pl.kernel,
        out_type=[jax.ShapeDtypeStruct((NP,), F32)] * 3
        + [jax.ShapeDtypeStruct((2 * EPAD,), jnp.int32),
           jax.ShapeDtypeStruct((2 * EPAD,), jnp.int32),
           jax.ShapeDtypeStruct((2 * EPAD,), F32),
           jax.ShapeDtypeStruct((2 * NS * 2 * L,), jnp.int32)],
        mesh=mesh,
        scratch_types=[
            pltpu.VMEM((HR, 128), jnp.int32),
            pltpu.VMEM((HR, 128), jnp.int32),
            pltpu.VMEM((HR, 128), F32),
            pltpu.VMEM((CAP + L,), jnp.int32),
            pltpu.VMEM((CAP + L,), jnp.int32),
            pltpu.VMEM((CAP + L,), F32),
            pltpu.VMEM((CAP + L,), jnp.int32),
            pltpu.VMEM((CAP + L,), jnp.int32),
            pltpu.VMEM((CAP + L,), F32),
            pltpu.VMEM((L,), jnp.int32),
            pltpu.VMEM((NP,), F32),
            pltpu.VMEM((npv,), F32),
            pltpu.VMEM((npv,), F32),
            pltpu.VMEM((npv,), F32),
            pltpu.VMEM((npv,), F32),
            pltpu.VMEM((L,), F32),
            pltpu.VMEM_SHARED((NS, NP), F32),
        ],
        compiler_params=pltpu.CompilerParams(needs_layout_passes=False),
    )
    def deg_kernel(src_hbm, dst_hbm, s_hbm, c_hbm,
                   dinv_hbm, cdinv_hbm, dinv2_hbm,
                   bsrc_hbm, bdst_hbm, bs_hbm, cnt_hbm,
                   srcv, dstv, sv, b0s, b0d, b0w, b1s, b1d, b1w, cntv,
                   degacc, tmpv, accv, uv, vv, cv, part_sh):
        sid = lax.axis_index("s")

        def zb(i, _):
            degacc[pl.ds(i * L, L)] = jnp.zeros((L,), F32)
            return 0
        lax.fori_loop(0, NP // L, zb, 0)

        pltpu.sync_copy(c_hbm, cv)

        for p in range(2):
            prow = sid * EROWS_T + p * HR
            pltpu.sync_copy(src_hbm.at[pl.ds(prow, HR)], srcv)
            pltpu.sync_copy(dst_hbm.at[pl.ds(prow, HR)], dstv)
            pltpu.sync_copy(s_hbm.at[pl.ds(prow, HR)], sv)

            def scat(k2, offs):
                off0, off1 = offs
                r = k2 // 8
                sl = pl.ds((k2 % 8) * L, L)
                dv = dstv[r, sl]
                wv = sv[r, sl]
                xv = srcv[r, sl]
                plsc.addupdate_scatter(degacc, [dv], wv)

                m0 = dv < NPH
                c0 = jnp.max(plsc.all_reduce_population_count(m0))
                plsc.store_compressed(b0s.at[pl.ds(off0, L)], xv, mask=m0)
                plsc.store_compressed(b0d.at[pl.ds(off0, L)], dv, mask=m0)
                plsc.store_compressed(b0w.at[pl.ds(off0, L)], wv, mask=m0)

                m1 = jnp.logical_not(m0)
                c1 = L - c0
                plsc.store_compressed(b1s.at[pl.ds(off1, L)], xv, mask=m1)
                plsc.store_compressed(b1d.at[pl.ds(off1, L)], dv - NPH, mask=m1)
                plsc.store_compressed(b1w.at[pl.ds(off1, L)], wv, mask=m1)
                return (off0 + c0, off1 + c1)
            off0, off1 = lax.fori_loop(
                0, HR * 8, scat, (jnp.int32(0), jnp.int32(0)))

            for b, (off, bsb, bdb, bwb) in enumerate(
                    [(off0, b0s, b0d, b0w), (off1, b1s, b1d, b1w)]):
                npad_v = (-off) % 512

                def padf(i, _):
                    o = off + i * L
                    bsb[pl.ds(o, L)] = jnp.zeros((L,), jnp.int32)
                    bdb[pl.ds(o, L)] = jnp.full((L,), NPH, jnp.int32)
                    bwb[pl.ds(o, L)] = jnp.zeros((L,), F32)
                    return 0
                lax.fori_loop(0, (npad_v + L - 1) // L, padf, 0)

                base = b * EPAD + (sid * EROWS_T + p * HR) * 128
                pltpu.sync_copy(bsb.at[pl.ds(0, CAP)],
                                bsrc_hbm.at[pl.ds(base, CAP)])
                pltpu.sync_copy(bdb.at[pl.ds(0, CAP)],
                                bdst_hbm.at[pl.ds(base, CAP)])
                pltpu.sync_copy(bwb.at[pl.ds(0, CAP)],
                                bs_hbm.at[pl.ds(base, CAP)])

                nch = (off + npad_v) // 512
                cntv[...] = jnp.full((L,), nch, jnp.int32)
                crow = ((b * NS + sid) * 2 + p) * L
                pltpu.sync_copy(cntv, cnt_hbm.at[pl.ds(crow, L)])

        pltpu.sync_copy(degacc, part_sh.at[sid])
        plsc.subcore_barrier()

        base = sid * npv
        pltpu.sync_copy(part_sh.at[0, pl.ds(base, npv)], accv)
        for t in range(1, NS):
            pltpu.sync_copy(part_sh.at[t, pl.ds(base, npv)], tmpv)

            def red(k, _):
                sl = pl.ds(k * L, L)
                accv[sl] += tmpv[sl]
                return 0
            lax.fori_loop(0, nv, red, 0)

        cvec = cv[...]

        def fin(k, _):
            sl = pl.ds(k * L, L)
            deg = 1.0 + cvec * accv[sl]
            y = _fast_rsqrt(deg)
            accv[sl] = y
            uv[sl] = cvec * y
            vv[sl] = y * y
            return 0
        lax.fori_loop(0, nv, fin, 0)

        pltpu.sync_copy(accv, dinv_hbm.at[pl.ds(base, npv)])
        pltpu.sync_copy(uv, cdinv_hbm.at[pl.ds(base, npv)])
        pltpu.sync_copy(vv, dinv2_hbm.at[pl.ds(base, npv)])

    return deg_kernel


def _make_conv_kernel(N, NP, EROWS_T):
    CH = 4
    CE = CH * 128
    NPH = NP // 2
    npv = NPH // NS
    HR = EROWS_T // 2
    CAP = HR * 128
    EPAD = NS * EROWS_T * 128
    mesh = plsc.VectorSubcoreMesh(
        core_axis_name="c", subcore_axis_name="s",
        num_cores=NC, num_subcores=NS)

    @functools.partial(
        pl.kernel,
        out_type=jax.ShapeDtypeStruct((NP, 128), F32),
        mesh=mesh,
        scratch_types=[
            pltpu.VMEM((CE,), jnp.int32),
            pltpu.VMEM((CE,), jnp.int32),
            pltpu.VMEM((CH, 128), jnp.int32),
            pltpu.VMEM((CE,), F32),
            pltpu.VMEM((CE, 128), F32),
            pltpu.VMEM((NP,), F32),
            pltpu.VMEM((L,), F32),
            pltpu.VMEM((L,), jnp.int32),
            pltpu.VMEM((64, 128), F32),
            pltpu.VMEM_SHARED((NPH + 64, 128), F32),
            pltpu.SemaphoreType.DMA,
        ],
        compiler_params=pltpu.CompilerParams(needs_layout_passes=False),
    )
    def conv_kernel(g_hbm, bsrc_hbm, bdst_hbm, bs_hbm, cnt_hbm, dinv_hbm,
                    out_hbm, srcv, dstv, dstv2, sv, rows, dinvv, wbuf, cntv,
                    zbuf, acc_sh, sem):
        cid = lax.axis_index("c")
        sid = lax.axis_index("s")

        def zb(i, _):
            zbuf[i // 8, pl.ds((i % 8) * L, L)] = jnp.zeros((L,), F32)
            return 0
        lax.fori_loop(0, 64 * 8, zb, 0)
        for t in range(npv // 64):
            pltpu.sync_copy(zbuf,
                            acc_sh.at[pl.ds(sid * npv + t * 64, 64)])

        pltpu.sync_copy(dinv_hbm, dinvv)
        plsc.subcore_barrier()

        for p in range(2):
            crow = ((cid * NS + sid) * 2 + p) * L
            pltpu.sync_copy(cnt_hbm.at[pl.ds(crow, L)], cntv)
            nch = jnp.max(cntv[...])
            base_e = cid * EPAD + (sid * EROWS_T + p * HR) * 128

            def chunk(kk, _):
                eb = base_e + kk * CE
                pltpu.sync_copy(bsrc_hbm.at[pl.ds(eb, CE)], srcv)
                pltpu.sync_copy(bdst_hbm.at[pl.ds(eb, CE)], dstv)
                pltpu.sync_copy(bs_hbm.at[pl.ds(eb, CE)], sv)

                def cpd(r, _):
                    for j in range(8):
                        dstv2[r, pl.ds(j * L, L)] = (
                            dstv[pl.ds(r * 128 + j * L, L)])
                    return 0
                lax.fori_loop(0, CH, cpd, 0)

                descs = [
                    pltpu.async_copy(
                        g_hbm.at[srcv.at[pl.ds(j * 128, 128)]],
                        rows.at[pl.ds(j * 128, 128)], sem)
                    for j in range(CH)
                ]
                for d in descs:
                    d.wait()

                def grp(k2, _):
                    sl = pl.ds(k2 * L, L)
                    svec = sv[sl]
                    srcvec = srcv[sl]
                    dg = plsc.load_gather(dinvv, [srcvec])
                    wbuf[...] = svec * dg
                    r0 = k2 * L
                    for r in range(L):
                        wb = plsc.load_gather(
                            wbuf, [jnp.full((L,), r, jnp.int32)])
                        for jj in range(8):
                            rows[r0 + r, pl.ds(jj * L, L)] *= wb
                    return 0
                lax.fori_loop(0, CE // L, grp, 0)

                for j in range(CH):
                    pltpu.sync_copy(rows.at[pl.ds(j * 128, 128)],
                                    acc_sh.at[dstv2.at[j]], add=True)
                return 0
            lax.fori_loop(0, nch, chunk, 0)

        plsc.subcore_barrier()
        pltpu.sync_copy(acc_sh.at[pl.ds(sid * npv, npv)],
                        out_hbm.at[pl.ds(cid * NPH + sid * npv, npv)])

    return conv_kernel


def kernel(x, edge_index, edge_attr, Wp, bp, W1, b1, W2, b2, Wo, bo):
    N, D = x.shape
    E, DE = edge_attr.shape
    H = W1.shape[1]
    assert D == 128 and H == 128 and DE == 16

    NP = -(-N // 2048) * 2048
    EROWS_T = -(-E // (NS * 128 * 4)) * 4
    EROWS = NS * EROWS_T
    EPAD = EROWS * 128

    src = edge_index[0]
    dst = edge_index[1]

    ER = E // 8
    ea2 = edge_attr.reshape(ER, 128)
    gmat = jnp.repeat(jnp.eye(8, dtype=F32), 16, axis=0)
    BE = 4000
    s8, ssq = pl.pallas_call(
        _edge_sum_body,
        grid=(ER // BE,),
        in_specs=[
            pl.BlockSpec((BE, 128), lambda i: (i, 0)),
            pl.BlockSpec((128, 8), lambda i: (0, 0)),
        ],
        out_specs=[
            pl.BlockSpec((BE, 8), lambda i: (i, 0)),
            pl.BlockSpec((1, 1), lambda i: (0, 0)),
        ],
        out_shape=[
            jax.ShapeDtypeStruct((ER, 8), F32),
            jax.ShapeDtypeStruct((1, 1), F32),
        ],
    )(ea2, gmat)

    c = 1.0 / jnp.maximum(jnp.sqrt(ssq[0, 0]), 1e-12)

    BN = 2000
    mlp_call = pl.pallas_call(
        _mlp_body,
        grid=(N // BN,),
        in_specs=[
            pl.BlockSpec((BN, D), lambda i: (i, 0)),
            pl.BlockSpec((D, H), lambda i: (0, 0)),
            pl.BlockSpec((1, H), lambda i: (0, 0)),
            pl.BlockSpec((H, H), lambda i: (0, 0)),
        ],
        out_specs=pl.BlockSpec((BN, H), lambda i: (i, 0)),
        out_shape=jax.ShapeDtypeStruct((N, H), F32),
    )
    hw1 = mlp_call(x, Wp, bp.reshape(1, H), W1)

    pad = EPAD - E
    padi = jnp.zeros((pad,), jnp.int32)
    src_p = jnp.concatenate([src, padi]).reshape(EROWS, 128)
    dst_p = jnp.concatenate([dst, jnp.full((pad,), N + 8, jnp.int32)]
                            ).reshape(EROWS, 128)
    s_p = jnp.concatenate([s8.reshape(E),
                           jnp.zeros((pad,), F32)]).reshape(EROWS, 128)
    cq = jnp.full((L,), c, F32)

    deg_kernel = _make_deg_kernel(NP, EROWS_T)
    dinv_f, cdinv_f, dinv2_f, bsrc, bdst, bs, cnts = deg_kernel(
        src_p, dst_p, s_p, cq)
    u_col = cdinv_f[:N].reshape(N, 1)
    v_col = dinv2_f[:N].reshape(N, 1)

    conv_kernel = _make_conv_kernel(N, NP, EROWS_T)

    ep_call = pl.pallas_call(
        _ep_body,
        grid=(N // BN,),
        in_specs=[
            pl.BlockSpec(memory_space=pltpu.SMEM),
            pl.BlockSpec((BN, 128), lambda i: (i, 0)),
            pl.BlockSpec((BN, 1), lambda i: (i, 0)),
            pl.BlockSpec((BN, 1), lambda i: (i, 0)),
            pl.BlockSpec((BN, H), lambda i: (i, 0)),
            pl.BlockSpec((1, H), lambda i: (0, 0)),
            pl.BlockSpec((H, H), lambda i: (0, 0)),
            pl.BlockSpec((1, H), lambda i: (0, 0)),
        ],
        out_specs=pl.BlockSpec((BN, H), lambda i: (i, 0)),
        out_shape=jax.ShapeDtypeStruct((N, H), F32),
    )

    def step(hw, ws):
        w_s, b_s, bout_s, flag_s = ws
        acc = conv_kernel(hw, bsrc, bdst, bs, cnts, dinv_f)
        hw_next = ep_call(flag_s, acc, u_col, v_col, hw, b_s, w_s, bout_s)
        return hw_next, 0.0

    ws = (
        jnp.stack([W2, Wo]),
        jnp.stack([b1.reshape(1, H), b2.reshape(1, H)]),
        jnp.stack([jnp.zeros((1, H), F32), bo.reshape(1, H)]),
        jnp.stack([jnp.ones((1, 1), F32), jnp.zeros((1, 1), F32)]),
    )
    out, _ = lax.scan(step, hw1, ws)
    return out


# speedup vs baseline: 3.1964x; 3.1964x over previous
import functools

import jax
import jax.numpy as jnp
from jax import lax
from jax.experimental import pallas as pl
from jax.experimental.pallas import tpu as pltpu
from jax.experimental.pallas import tpu_sc as plsc

NC = 2
NS = 16
L = 16
NW = NC * NS

F32 = jnp.float32


def _edge_sum_body(ea_ref, g_ref, s_ref, ssq_ref):
    i = pl.program_id(0)
    sb = jnp.dot(ea_ref[...], g_ref[...], preferred_element_type=F32)
    s_ref[...] = sb

    @pl.when(i == 0)
    def _():
        ssq_ref[...] = jnp.zeros_like(ssq_ref)

    ssq_ref[...] += jnp.sum(sb * sb).reshape(1, 1)


def _mlp_body(x_ref, wp_ref, bp_ref, w1_ref, o_ref):
    h = jnp.dot(x_ref[...], wp_ref[...], preferred_element_type=F32)
    h = jnp.maximum(h + bp_ref[...], 0.0)
    o_ref[...] = jnp.dot(h, w1_ref[...], preferred_element_type=F32)


def _ep_body(flag_ref, a_ref, u_ref, v_ref, hw_ref, b_ref, w_ref, bout_ref,
             o_ref):
    asum = a_ref[...]
    h = asum * u_ref[...] + hw_ref[...] * v_ref[...] + b_ref[...]
    f = flag_ref[0, 0]
    h = f * jnp.maximum(h, 0.0) + (1.0 - f) * h
    o_ref[...] = (jnp.dot(h, w_ref[...], preferred_element_type=F32)
                  + bout_ref[...])


def _fast_rsqrt(x):
    i = plsc.bitcast(x, jnp.int32)
    i = jnp.int32(0x5F3759DF) - lax.shift_right_arithmetic(i, 1)
    y = plsc.bitcast(i, F32)
    half = x * 0.5
    for _ in range(4):
        y = y * (1.5 - half * y * y)
    return y


def _make_deg_kernel(NP, EROWS_T):
    npv = NP // NS
    nv = npv // L
    NPH = NP // 2
    HR = EROWS_T // 2
    CAP = HR * 128
    EPAD = NS * EROWS_T * 128
    mesh = plsc.VectorSubcoreMesh(
        core_axis_name="c", subcore_axis_name="s",
        num_cores=1, num_subcores=NS)

    @functools.partial(
        pl.kernel,
        out_type=[jax.ShapeDtypeStruct((NP,), F32)] * 3
        + [jax.ShapeDtypeStruct((2 * EPAD,), jnp.int32),
           jax.ShapeDtypeStruct((2 * EPAD,), jnp.int32),
           jax.ShapeDtypeStruct((2 * EPAD,), F32),
           jax.ShapeDtypeStruct((2 * NS * 2 * L,), jnp.int32)],
        mesh=mesh,
        scratch_types=[
            pltpu.VMEM((HR, 128), jnp.int32),
            pltpu.VMEM((HR, 128), jnp.int32),
            pltpu.VMEM((HR, 128), F32),
            pltpu.VMEM((CAP + L,), jnp.int32),
            pltpu.VMEM((CAP + L,), jnp.int32),
            pltpu.VMEM((CAP + L,), F32),
            pltpu.VMEM((CAP + L,), jnp.int32),
            pltpu.VMEM((CAP + L,), jnp.int32),
            pltpu.VMEM((CAP + L,), F32),
            pltpu.VMEM((L,), jnp.int32),
            pltpu.VMEM((NP,), F32),
            pltpu.VMEM((npv,), F32),
            pltpu.VMEM((npv,), F32),
            pltpu.VMEM((npv,), F32),
            pltpu.VMEM((npv,), F32),
            pltpu.VMEM((L,), F32),
            pltpu.VMEM_SHARED((NS, NP), F32),
        ],
        compiler_params=pltpu.CompilerParams(needs_layout_passes=False),
    )
    def deg_kernel(src_hbm, dst_hbm, s_hbm, c_hbm,
                   dinv_hbm, cdinv_hbm, dinv2_hbm,
                   bsrc_hbm, bdst_hbm, bs_hbm, cnt_hbm,
                   srcv, dstv, sv, b0s, b0d, b0w, b1s, b1d, b1w, cntv,
                   degacc, tmpv, accv, uv, vv, cv, part_sh):
        sid = lax.axis_index("s")

        def zb(i, _):
            degacc[pl.ds(i * L, L)] = jnp.zeros((L,), F32)
            return 0
        lax.fori_loop(0, NP // L, zb, 0)

        pltpu.sync_copy(c_hbm, cv)

        for p in range(2):
            prow = sid * EROWS_T + p * HR
            pltpu.sync_copy(src_hbm.at[pl.ds(prow, HR)], srcv)
            pltpu.sync_copy(dst_hbm.at[pl.ds(prow, HR)], dstv)
            pltpu.sync_copy(s_hbm.at[pl.ds(prow, HR)], sv)

            def scat(k2, offs):
                off0, off1 = offs
                r = k2 // 8
                sl = pl.ds((k2 % 8) * L, L)
                dv = dstv[r, sl]
                wv = sv[r, sl]
                xv = srcv[r, sl]
                plsc.addupdate_scatter(degacc, [dv], wv)

                m0 = dv < NPH
                c0 = jnp.max(plsc.all_reduce_population_count(m0))
                plsc.store_compressed(b0s.at[pl.ds(off0, L)], xv, mask=m0)
                plsc.store_compressed(b0d.at[pl.ds(off0, L)], dv, mask=m0)
                plsc.store_compressed(b0w.at[pl.ds(off0, L)], wv, mask=m0)

                m1 = jnp.logical_not(m0)
                c1 = L - c0
                plsc.store_compressed(b1s.at[pl.ds(off1, L)], xv, mask=m1)
                plsc.store_compressed(b1d.at[pl.ds(off1, L)], dv - NPH, mask=m1)
                plsc.store_compressed(b1w.at[pl.ds(off1, L)], wv, mask=m1)
                return (off0 + c0, off1 + c1)
            off0, off1 = lax.fori_loop(
                0, HR * 8, scat, (jnp.int32(0), jnp.int32(0)))

            for b, (off, bsb, bdb, bwb) in enumerate(
                    [(off0, b0s, b0d, b0w), (off1, b1s, b1d, b1w)]):
                npad_v = (-off) % 512

                def padf(i, _):
                    o = off + i * L
                    spread = (jnp.full((L,), o & 4095, jnp.int32)
                              + lax.iota(jnp.int32, L))
                    bsb[pl.ds(o, L)] = spread
                    bdb[pl.ds(o, L)] = spread
                    bwb[pl.ds(o, L)] = jnp.zeros((L,), F32)
                    return 0
                lax.fori_loop(0, (npad_v + L - 1) // L, padf, 0)

                base = b * EPAD + (sid * EROWS_T + p * HR) * 128
                pltpu.sync_copy(bsb.at[pl.ds(0, CAP)],
                                bsrc_hbm.at[pl.ds(base, CAP)])
                pltpu.sync_copy(bdb.at[pl.ds(0, CAP)],
                                bdst_hbm.at[pl.ds(base, CAP)])
                pltpu.sync_copy(bwb.at[pl.ds(0, CAP)],
                                bs_hbm.at[pl.ds(base, CAP)])

                nch = (off + npad_v) // 512
                cntv[...] = jnp.full((L,), nch, jnp.int32)
                crow = ((b * NS + sid) * 2 + p) * L
                pltpu.sync_copy(cntv, cnt_hbm.at[pl.ds(crow, L)])

        pltpu.sync_copy(degacc, part_sh.at[sid])
        plsc.subcore_barrier()

        base = sid * npv
        pltpu.sync_copy(part_sh.at[0, pl.ds(base, npv)], accv)
        for t in range(1, NS):
            pltpu.sync_copy(part_sh.at[t, pl.ds(base, npv)], tmpv)

            def red(k, _):
                sl = pl.ds(k * L, L)
                accv[sl] += tmpv[sl]
                return 0
            lax.fori_loop(0, nv, red, 0)

        cvec = cv[...]

        def fin(k, _):
            sl = pl.ds(k * L, L)
            deg = 1.0 + cvec * accv[sl]
            y = _fast_rsqrt(deg)
            accv[sl] = y
            uv[sl] = cvec * y
            vv[sl] = y * y
            return 0
        lax.fori_loop(0, nv, fin, 0)

        pltpu.sync_copy(accv, dinv_hbm.at[pl.ds(base, npv)])
        pltpu.sync_copy(uv, cdinv_hbm.at[pl.ds(base, npv)])
        pltpu.sync_copy(vv, dinv2_hbm.at[pl.ds(base, npv)])

    return deg_kernel


def _make_conv_kernel(N, NP, EROWS_T):
    CH = 4
    CE = CH * 128
    NPH = NP // 2
    npv = NPH // NS
    HR = EROWS_T // 2
    CAP = HR * 128
    EPAD = NS * EROWS_T * 128
    mesh = plsc.VectorSubcoreMesh(
        core_axis_name="c", subcore_axis_name="s",
        num_cores=NC, num_subcores=NS)

    @functools.partial(
        pl.kernel,
        out_type=jax.ShapeDtypeStruct((NP, 128), F32),
        mesh=mesh,
        scratch_types=[
            pltpu.VMEM((CE,), jnp.int32),
            pltpu.VMEM((CE,), jnp.int32),
            pltpu.VMEM((CH, 128), jnp.int32),
            pltpu.VMEM((CE,), F32),
            pltpu.VMEM((CE, 128), F32),
            pltpu.VMEM((NP,), F32),
            pltpu.VMEM((L,), F32),
            pltpu.VMEM((L,), jnp.int32),
            pltpu.VMEM((64, 128), F32),
            pltpu.VMEM_SHARED((NPH + 64, 128), F32),
            pltpu.SemaphoreType.DMA,
        ],
        compiler_params=pltpu.CompilerParams(needs_layout_passes=False),
    )
    def conv_kernel(g_hbm, bsrc_hbm, bdst_hbm, bs_hbm, cnt_hbm, dinv_hbm,
                    out_hbm, srcv, dstv, dstv2, sv, rows, dinvv, wbuf, cntv,
                    zbuf, acc_sh, sem):
        cid = lax.axis_index("c")
        sid = lax.axis_index("s")

        def zb(i, _):
            zbuf[i // 8, pl.ds((i % 8) * L, L)] = jnp.zeros((L,), F32)
            return 0
        lax.fori_loop(0, 64 * 8, zb, 0)
        for t in range(npv // 64):
            pltpu.sync_copy(zbuf,
                            acc_sh.at[pl.ds(sid * npv + t * 64, 64)])

        pltpu.sync_copy(dinv_hbm, dinvv)
        plsc.subcore_barrier()

        for p in range(2):
            crow = ((cid * NS + sid) * 2 + p) * L
            pltpu.sync_copy(cnt_hbm.at[pl.ds(crow, L)], cntv)
            nch = jnp.max(cntv[...])
            base_e = cid * EPAD + (sid * EROWS_T + p * HR) * 128

            def chunk(kk, _):
                eb = base_e + kk * CE
                pltpu.sync_copy(bsrc_hbm.at[pl.ds(eb, CE)], srcv)
                pltpu.sync_copy(bdst_hbm.at[pl.ds(eb, CE)], dstv)
                pltpu.sync_copy(bs_hbm.at[pl.ds(eb, CE)], sv)

                def cpd(r, _):
                    for j in range(8):
                        dstv2[r, pl.ds(j * L, L)] = (
                            dstv[pl.ds(r * 128 + j * L, L)])
                    return 0
                lax.fori_loop(0, CH, cpd, 0)

                descs = [
                    pltpu.async_copy(
                        g_hbm.at[srcv.at[pl.ds(j * 128, 128)]],
                        rows.at[pl.ds(j * 128, 128)], sem)
                    for j in range(CH)
                ]
                for d in descs:
                    d.wait()

                def grp(k2, _):
                    sl = pl.ds(k2 * L, L)
                    svec = sv[sl]
                    srcvec = srcv[sl]
                    dg = plsc.load_gather(dinvv, [srcvec])
                    wbuf[...] = svec * dg
                    r0 = k2 * L
                    for r in range(L):
                        wb = plsc.load_gather(
                            wbuf, [jnp.full((L,), r, jnp.int32)])
                        for jj in range(8):
                            rows[r0 + r, pl.ds(jj * L, L)] *= wb
                    return 0
                lax.fori_loop(0, CE // L, grp, 0)

                for j in range(CH):
                    pltpu.sync_copy(rows.at[pl.ds(j * 128, 128)],
                                    acc_sh.at[dstv2.at[j]], add=True)
                return 0
            lax.fori_loop(0, nch, chunk, 0)

        plsc.subcore_barrier()
        pltpu.sync_copy(acc_sh.at[pl.ds(sid * npv, npv)],
                        out_hbm.at[pl.ds(cid * NPH + sid * npv, npv)])

    return conv_kernel


def kernel(x, edge_index, edge_attr, Wp, bp, W1, b1, W2, b2, Wo, bo):
    N, D = x.shape
    E, DE = edge_attr.shape
    H = W1.shape[1]
    assert D == 128 and H == 128 and DE == 16

    NP = -(-N // 2048) * 2048
    EROWS_T = -(-E // (NS * 128 * 4)) * 4
    EROWS = NS * EROWS_T
    EPAD = EROWS * 128

    src = edge_index[0]
    dst = edge_index[1]

    ER = E // 8
    ea2 = edge_attr.reshape(ER, 128)
    gmat = jnp.repeat(jnp.eye(8, dtype=F32), 16, axis=0)
    BE = 4000
    s8, ssq = pl.pallas_call(
        _edge_sum_body,
        grid=(ER // BE,),
        in_specs=[
            pl.BlockSpec((BE, 128), lambda i: (i, 0)),
            pl.BlockSpec((128, 8), lambda i: (0, 0)),
        ],
        out_specs=[
            pl.BlockSpec((BE, 8), lambda i: (i, 0)),
            pl.BlockSpec((1, 1), lambda i: (0, 0)),
        ],
        out_shape=[
            jax.ShapeDtypeStruct((ER, 8), F32),
            jax.ShapeDtypeStruct((1, 1), F32),
        ],
    )(ea2, gmat)

    c = 1.0 / jnp.maximum(jnp.sqrt(ssq[0, 0]), 1e-12)

    BN = 2000
    mlp_call = pl.pallas_call(
        _mlp_body,
        grid=(N // BN,),
        in_specs=[
            pl.BlockSpec((BN, D), lambda i: (i, 0)),
            pl.BlockSpec((D, H), lambda i: (0, 0)),
            pl.BlockSpec((1, H), lambda i: (0, 0)),
            pl.BlockSpec((H, H), lambda i: (0, 0)),
        ],
        out_specs=pl.BlockSpec((BN, H), lambda i: (i, 0)),
        out_shape=jax.ShapeDtypeStruct((N, H), F32),
    )
    hw1 = mlp_call(x, Wp, bp.reshape(1, H), W1)

    pad = EPAD - E
    padi = jnp.arange(pad, dtype=jnp.int32) % N
    src_p = jnp.concatenate([src, padi]).reshape(EROWS, 128)
    dst_p = jnp.concatenate([dst, padi]).reshape(EROWS, 128)
    s_p = jnp.concatenate([s8.reshape(E),
                           jnp.zeros((pad,), F32)]).reshape(EROWS, 128)
    cq = jnp.full((L,), c, F32)

    deg_kernel = _make_deg_kernel(NP, EROWS_T)
    dinv_f, cdinv_f, dinv2_f, bsrc, bdst, bs, cnts = deg_kernel(
        src_p, dst_p, s_p, cq)
    u_col = cdinv_f[:N].reshape(N, 1)
    v_col = dinv2_f[:N].reshape(N, 1)

    conv_kernel = _make_conv_kernel(N, NP, EROWS_T)

    ep_call = pl.pallas_call(
        _ep_body,
        grid=(N // BN,),
        in_specs=[
            pl.BlockSpec(memory_space=pltpu.SMEM),
            pl.BlockSpec((BN, 128), lambda i: (i, 0)),
            pl.BlockSpec((BN, 1), lambda i: (i, 0)),
            pl.BlockSpec((BN, 1), lambda i: (i, 0)),
            pl.BlockSpec((BN, H), lambda i: (i, 0)),
            pl.BlockSpec((1, H), lambda i: (0, 0)),
            pl.BlockSpec((H, H), lambda i: (0, 0)),
            pl.BlockSpec((1, H), lambda i: (0, 0)),
        ],
        out_specs=pl.BlockSpec((BN, H), lambda i: (i, 0)),
        out_shape=jax.ShapeDtypeStruct((N, H), F32),
    )

    def step(hw, ws):
        w_s, b_s, bout_s, flag_s = ws
        acc = conv_kernel(hw, bsrc, bdst, bs, cnts, dinv_f)
        hw_next = ep_call(flag_s, acc, u_col, v_col, hw, b_s, w_s, bout_s)
        return hw_next, 0.0

    ws = (
        jnp.stack([W2, Wo]),
        jnp.stack([b1.reshape(1, H), b2.reshape(1, H)]),
        jnp.stack([jnp.zeros((1, H), F32), bo.reshape(1, H)]),
        jnp.stack([jnp.ones((1, 1), F32), jnp.zeros((1, 1), F32)]),
    )
    out, _ = lax.scan(step, hw1, ws)
    return out


# speedup vs baseline: 3.5950x; 1.1247x over previous
import functools

import jax
import jax.numpy as jnp
from jax import lax
from jax.experimental import pallas as pl
from jax.experimental.pallas import tpu as pltpu
from jax.experimental.pallas import tpu_sc as plsc

NC = 2
NS = 16
L = 16
NW = NC * NS

F32 = jnp.float32


def _edge_sum_body(ea_ref, g_ref, s_ref, ssq_ref):
    i = pl.program_id(0)
    sb = jnp.dot(ea_ref[...], g_ref[...], preferred_element_type=F32)
    s_ref[...] = sb

    @pl.when(i == 0)
    def _():
        ssq_ref[...] = jnp.zeros_like(ssq_ref)

    ssq_ref[...] += jnp.sum(sb * sb).reshape(1, 1)


def _mlp_body(x_ref, wp_ref, bp_ref, w1_ref, o_ref):
    h = jnp.dot(x_ref[...], wp_ref[...], preferred_element_type=F32)
    h = jnp.maximum(h + bp_ref[...], 0.0)
    o_ref[...] = jnp.dot(h, w1_ref[...], preferred_element_type=F32)


def _ep_body(flag_ref, a_ref, u_ref, v_ref, hw_ref, b_ref, w_ref, bout_ref,
             o_ref):
    asum = a_ref[...]
    h = asum * u_ref[...] + hw_ref[...] * v_ref[...] + b_ref[...]
    f = flag_ref[0, 0]
    h = f * jnp.maximum(h, 0.0) + (1.0 - f) * h
    o_ref[...] = (jnp.dot(h, w_ref[...], preferred_element_type=F32)
                  + bout_ref[...])


def _fast_rsqrt(x):
    i = plsc.bitcast(x, jnp.int32)
    i = jnp.int32(0x5F3759DF) - lax.shift_right_arithmetic(i, 1)
    y = plsc.bitcast(i, F32)
    half = x * 0.5
    for _ in range(4):
        y = y * (1.5 - half * y * y)
    return y


def _make_deg_kernel(NP, EROWS_T):
    npv = NP // NS
    nv = npv // L
    NPH = NP // 2
    HR = EROWS_T // 2
    CAP = HR * 128
    EPAD = NS * EROWS_T * 128
    mesh = plsc.VectorSubcoreMesh(
        core_axis_name="c", subcore_axis_name="s",
        num_cores=1, num_subcores=NS)

    @functools.partial(
        pl.kernel,
        out_type=[jax.ShapeDtypeStruct((NP,), F32)] * 3
        + [jax.ShapeDtypeStruct((2 * EPAD,), jnp.int32),
           jax.ShapeDtypeStruct((2 * EPAD,), jnp.int32),
           jax.ShapeDtypeStruct((2 * EPAD,), F32),
           jax.ShapeDtypeStruct((2 * NS * 2 * L,), jnp.int32)],
        mesh=mesh,
        scratch_types=[
            pltpu.VMEM((HR, 128), jnp.int32),
            pltpu.VMEM((HR, 128), jnp.int32),
            pltpu.VMEM((HR, 128), F32),
            pltpu.VMEM((CAP + L,), jnp.int32),
            pltpu.VMEM((CAP + L,), jnp.int32),
            pltpu.VMEM((CAP + L,), F32),
            pltpu.VMEM((CAP + L,), jnp.int32),
            pltpu.VMEM((CAP + L,), jnp.int32),
            pltpu.VMEM((CAP + L,), F32),
            pltpu.VMEM((L,), jnp.int32),
            pltpu.VMEM((NP,), F32),
            pltpu.VMEM((npv,), F32),
            pltpu.VMEM((npv,), F32),
            pltpu.VMEM((npv,), F32),
            pltpu.VMEM((npv,), F32),
            pltpu.VMEM((L,), F32),
            pltpu.VMEM_SHARED((NS, NP), F32),
        ],
        compiler_params=pltpu.CompilerParams(needs_layout_passes=False),
    )
    def deg_kernel(src_hbm, dst_hbm, s_hbm, c_hbm,
                   dinv_hbm, cdinv_hbm, dinv2_hbm,
                   bsrc_hbm, bdst_hbm, bs_hbm, cnt_hbm,
                   srcv, dstv, sv, b0s, b0d, b0w, b1s, b1d, b1w, cntv,
                   degacc, tmpv, accv, uv, vv, cv, part_sh):
        sid = lax.axis_index("s")

        def zb(i, _):
            degacc[pl.ds(i * L, L)] = jnp.zeros((L,), F32)
            return 0
        lax.fori_loop(0, NP // L, zb, 0)

        pltpu.sync_copy(c_hbm, cv)

        for p in range(2):
            prow = sid * EROWS_T + p * HR
            pltpu.sync_copy(src_hbm.at[pl.ds(prow, HR)], srcv)
            pltpu.sync_copy(dst_hbm.at[pl.ds(prow, HR)], dstv)
            pltpu.sync_copy(s_hbm.at[pl.ds(prow, HR)], sv)

            def scat(k2, offs):
                off0, off1 = offs
                r = k2 // 8
                sl = pl.ds((k2 % 8) * L, L)
                dv = dstv[r, sl]
                wv = sv[r, sl]
                xv = srcv[r, sl]
                plsc.addupdate_scatter(degacc, [dv], wv)

                m0 = dv < NPH
                c0 = jnp.max(plsc.all_reduce_population_count(m0))
                plsc.store_compressed(b0s.at[pl.ds(off0, L)], xv, mask=m0)
                plsc.store_compressed(b0d.at[pl.ds(off0, L)], dv, mask=m0)
                plsc.store_compressed(b0w.at[pl.ds(off0, L)], wv, mask=m0)

                m1 = jnp.logical_not(m0)
                c1 = L - c0
                plsc.store_compressed(b1s.at[pl.ds(off1, L)], xv, mask=m1)
                plsc.store_compressed(b1d.at[pl.ds(off1, L)], dv - NPH, mask=m1)
                plsc.store_compressed(b1w.at[pl.ds(off1, L)], wv, mask=m1)
                return (off0 + c0, off1 + c1)
            off0, off1 = lax.fori_loop(
                0, HR * 8, scat, (jnp.int32(0), jnp.int32(0)))

            for b, (off, bsb, bdb, bwb) in enumerate(
                    [(off0, b0s, b0d, b0w), (off1, b1s, b1d, b1w)]):
                npad_v = (-off) % 256

                def padf(i, _):
                    o = off + i * L
                    spread = (jnp.full((L,), o & 4095, jnp.int32)
                              + lax.iota(jnp.int32, L))
                    bsb[pl.ds(o, L)] = spread
                    bdb[pl.ds(o, L)] = spread
                    bwb[pl.ds(o, L)] = jnp.zeros((L,), F32)
                    return 0
                lax.fori_loop(0, (npad_v + L - 1) // L, padf, 0)

                base = b * EPAD + (sid * EROWS_T + p * HR) * 128
                pltpu.sync_copy(bsb.at[pl.ds(0, CAP)],
                                bsrc_hbm.at[pl.ds(base, CAP)])
                pltpu.sync_copy(bdb.at[pl.ds(0, CAP)],
                                bdst_hbm.at[pl.ds(base, CAP)])
                pltpu.sync_copy(bwb.at[pl.ds(0, CAP)],
                                bs_hbm.at[pl.ds(base, CAP)])

                nch = (off + npad_v) // 256
                cntv[...] = jnp.full((L,), nch, jnp.int32)
                crow = ((b * NS + sid) * 2 + p) * L
                pltpu.sync_copy(cntv, cnt_hbm.at[pl.ds(crow, L)])

        pltpu.sync_copy(degacc, part_sh.at[sid])
        plsc.subcore_barrier()

        base = sid * npv
        pltpu.sync_copy(part_sh.at[0, pl.ds(base, npv)], accv)
        for t in range(1, NS):
            pltpu.sync_copy(part_sh.at[t, pl.ds(base, npv)], tmpv)

            def red(k, _):
                sl = pl.ds(k * L, L)
                accv[sl] += tmpv[sl]
                return 0
            lax.fori_loop(0, nv, red, 0)

        cvec = cv[...]

        def fin(k, _):
            sl = pl.ds(k * L, L)
            deg = 1.0 + cvec * accv[sl]
            y = _fast_rsqrt(deg)
            accv[sl] = y
            uv[sl] = cvec * y
            vv[sl] = y * y
            return 0
        lax.fori_loop(0, nv, fin, 0)

        pltpu.sync_copy(accv, dinv_hbm.at[pl.ds(base, npv)])
        pltpu.sync_copy(uv, cdinv_hbm.at[pl.ds(base, npv)])
        pltpu.sync_copy(vv, dinv2_hbm.at[pl.ds(base, npv)])

    return deg_kernel


def _make_conv_kernel(N, NP, EROWS_T):
    CH = 2
    CE = CH * 128
    NPH = NP // 2
    npv = NPH // NS
    HR = EROWS_T // 2
    CAP = HR * 128
    EPAD = NS * EROWS_T * 128
    mesh = plsc.VectorSubcoreMesh(
        core_axis_name="c", subcore_axis_name="s",
        num_cores=NC, num_subcores=NS)

    @functools.partial(
        pl.kernel,
        out_type=jax.ShapeDtypeStruct((NP, 128), F32),
        mesh=mesh,
        scratch_types=[
            pltpu.VMEM((2 * CE,), jnp.int32),
            pltpu.VMEM((2 * CE,), jnp.int32),
            pltpu.VMEM((2 * CH, 128), jnp.int32),
            pltpu.VMEM((2 * CE,), F32),
            pltpu.VMEM((2 * CE, 128), F32),
            pltpu.VMEM((NP,), F32),
            pltpu.VMEM((L,), F32),
            pltpu.VMEM((L,), jnp.int32),
            pltpu.VMEM((64, 128), F32),
            pltpu.VMEM_SHARED((NPH + 64, 128), F32),
            pltpu.SemaphoreType.DMA,
            pltpu.SemaphoreType.DMA,
        ],
        compiler_params=pltpu.CompilerParams(needs_layout_passes=False),
    )
    def conv_kernel(g_hbm, bsrc_hbm, bdst_hbm, bs_hbm, cnt_hbm, dinv_hbm,
                    out_hbm, srcv, dstv, dstv2, sv, rows, dinvv, wbuf, cntv,
                    zbuf, acc_sh, sem_g, sem_s):
        cid = lax.axis_index("c")
        sid = lax.axis_index("s")

        def zb(i, _):
            zbuf[i // 8, pl.ds((i % 8) * L, L)] = jnp.zeros((L,), F32)
            return 0
        lax.fori_loop(0, 64 * 8, zb, 0)
        for t in range(npv // 64):
            pltpu.sync_copy(zbuf,
                            acc_sh.at[pl.ds(sid * npv + t * 64, 64)])

        pltpu.sync_copy(dinv_hbm, dinvv)
        plsc.subcore_barrier()

        def stage(kk, base_e):
            b = kk & 1
            bo = b * CE
            eb = base_e + kk * CE
            pltpu.sync_copy(bsrc_hbm.at[pl.ds(eb, CE)],
                            srcv.at[pl.ds(bo, CE)])
            pltpu.sync_copy(bdst_hbm.at[pl.ds(eb, CE)],
                            dstv.at[pl.ds(bo, CE)])
            pltpu.sync_copy(bs_hbm.at[pl.ds(eb, CE)], sv.at[pl.ds(bo, CE)])

            def cpd(r, _):
                for j in range(8):
                    dstv2[b * CH + r, pl.ds(j * L, L)] = (
                        dstv[pl.ds(bo + r * 128 + j * L, L)])
                return 0
            lax.fori_loop(0, CH, cpd, 0)

            for j in range(CH):
                pltpu.async_copy(
                    g_hbm.at[srcv.at[pl.ds(bo + j * 128, 128)]],
                    rows.at[pl.ds(bo + j * 128, 128)], sem_g)

        for p in range(2):
            crow = ((cid * NS + sid) * 2 + p) * L
            pltpu.sync_copy(cnt_hbm.at[pl.ds(crow, L)], cntv)
            nch = jnp.max(cntv[...])
            base_e = cid * EPAD + (sid * EROWS_T + p * HR) * 128

            @pl.when(nch > 0)
            def _():
                stage(0, base_e)

            def chunk(kk, _):
                b = kk & 1
                bo = b * CE

                @pl.when(kk >= 1)
                def _():
                    pltpu.make_async_copy(
                        rows.at[pl.ds(0, CE)], acc_sh.at[pl.ds(0, CE)],
                        sem_s).wait()

                pltpu.make_async_copy(
                    g_hbm.at[pl.ds(0, CE)], rows.at[pl.ds(bo, CE)],
                    sem_g).wait()

                @pl.when(kk + 1 < nch)
                def _():
                    stage(kk + 1, base_e)

                def grp(k2, _):
                    sl = pl.ds(bo + k2 * L, L)
                    svec = sv[sl]
                    srcvec = srcv[sl]
                    dg = plsc.load_gather(dinvv, [srcvec])
                    wbuf[...] = svec * dg
                    r0 = bo + k2 * L
                    for r in range(L):
                        wb = plsc.load_gather(
                            wbuf, [jnp.full((L,), r, jnp.int32)])
                        for jj in range(8):
                            rows[r0 + r, pl.ds(jj * L, L)] *= wb
                    return 0
                lax.fori_loop(0, CE // L, grp, 0)

                for j in range(CH):
                    pltpu.async_copy(rows.at[pl.ds(bo + j * 128, 128)],
                                     acc_sh.at[dstv2.at[b * CH + j]], sem_s,
                                     add=True)
                return 0
            lax.fori_loop(0, nch, chunk, 0)

            @pl.when(nch >= 1)
            def _():
                pltpu.make_async_copy(
                    rows.at[pl.ds(0, CE)], acc_sh.at[pl.ds(0, CE)],
                    sem_s).wait()

        plsc.subcore_barrier()
        pltpu.sync_copy(acc_sh.at[pl.ds(sid * npv, npv)],
                        out_hbm.at[pl.ds(cid * NPH + sid * npv, npv)])

    return conv_kernel


def kernel(x, edge_index, edge_attr, Wp, bp, W1, b1, W2, b2, Wo, bo):
    N, D = x.shape
    E, DE = edge_attr.shape
    H = W1.shape[1]
    assert D == 128 and H == 128 and DE == 16

    NP = -(-N // 2048) * 2048
    EROWS_T = -(-E // (NS * 128 * 4)) * 4
    EROWS = NS * EROWS_T
    EPAD = EROWS * 128

    src = edge_index[0]
    dst = edge_index[1]

    ER = E // 8
    ea2 = edge_attr.reshape(ER, 128)
    gmat = jnp.repeat(jnp.eye(8, dtype=F32), 16, axis=0)
    BE = 4000
    s8, ssq = pl.pallas_call(
        _edge_sum_body,
        grid=(ER // BE,),
        in_specs=[
            pl.BlockSpec((BE, 128), lambda i: (i, 0)),
            pl.BlockSpec((128, 8), lambda i: (0, 0)),
        ],
        out_specs=[
            pl.BlockSpec((BE, 8), lambda i: (i, 0)),
            pl.BlockSpec((1, 1), lambda i: (0, 0)),
        ],
        out_shape=[
            jax.ShapeDtypeStruct((ER, 8), F32),
            jax.ShapeDtypeStruct((1, 1), F32),
        ],
    )(ea2, gmat)

    c = 1.0 / jnp.maximum(jnp.sqrt(ssq[0, 0]), 1e-12)

    BN = 2000
    mlp_call = pl.pallas_call(
        _mlp_body,
        grid=(N // BN,),
        in_specs=[
            pl.BlockSpec((BN, D), lambda i: (i, 0)),
            pl.BlockSpec((D, H), lambda i: (0, 0)),
            pl.BlockSpec((1, H), lambda i: (0, 0)),
            pl.BlockSpec((H, H), lambda i: (0, 0)),
        ],
        out_specs=pl.BlockSpec((BN, H), lambda i: (i, 0)),
        out_shape=jax.ShapeDtypeStruct((N, H), F32),
    )
    hw1 = mlp_call(x, Wp, bp.reshape(1, H), W1)

    pad = EPAD - E
    padi = jnp.arange(pad, dtype=jnp.int32) % N
    src_p = jnp.concatenate([src, padi]).reshape(EROWS, 128)
    dst_p = jnp.concatenate([dst, padi]).reshape(EROWS, 128)
    s_p = jnp.concatenate([s8.reshape(E),
                           jnp.zeros((pad,), F32)]).reshape(EROWS, 128)
    cq = jnp.full((L,), c, F32)

    deg_kernel = _make_deg_kernel(NP, EROWS_T)
    dinv_f, cdinv_f, dinv2_f, bsrc, bdst, bs, cnts = deg_kernel(
        src_p, dst_p, s_p, cq)
    u_col = cdinv_f[:N].reshape(N, 1)
    v_col = dinv2_f[:N].reshape(N, 1)

    conv_kernel = _make_conv_kernel(N, NP, EROWS_T)

    ep_call = pl.pallas_call(
        _ep_body,
        grid=(N // BN,),
        in_specs=[
            pl.BlockSpec(memory_space=pltpu.SMEM),
            pl.BlockSpec((BN, 128), lambda i: (i, 0)),
            pl.BlockSpec((BN, 1), lambda i: (i, 0)),
            pl.BlockSpec((BN, 1), lambda i: (i, 0)),
            pl.BlockSpec((BN, H), lambda i: (i, 0)),
            pl.BlockSpec((1, H), lambda i: (0, 0)),
            pl.BlockSpec((H, H), lambda i: (0, 0)),
            pl.BlockSpec((1, H), lambda i: (0, 0)),
        ],
        out_specs=pl.BlockSpec((BN, H), lambda i: (i, 0)),
        out_shape=jax.ShapeDtypeStruct((N, H), F32),
    )

    def step(hw, ws):
        w_s, b_s, bout_s, flag_s = ws
        acc = conv_kernel(hw, bsrc, bdst, bs, cnts, dinv_f)
        hw_next = ep_call(flag_s, acc, u_col, v_col, hw, b_s, w_s, bout_s)
        return hw_next, 0.0

    ws = (
        jnp.stack([W2, Wo]),
        jnp.stack([b1.reshape(1, H), b2.reshape(1, H)]),
        jnp.stack([jnp.zeros((1, H), F32), bo.reshape(1, H)]),
        jnp.stack([jnp.ones((1, 1), F32), jnp.zeros((1, 1), F32)]),
    )
    out, _ = lax.scan(step, hw1, ws)
    return out


# speedup vs baseline: 4.0337x; 1.1220x over previous
import functools

import jax
import jax.numpy as jnp
from jax import lax
from jax.experimental import pallas as pl
from jax.experimental.pallas import tpu as pltpu
from jax.experimental.pallas import tpu_sc as plsc

NC = 2
NS = 16
L = 16
NW = NC * NS

F32 = jnp.float32


def _edge_sum_body(ea_ref, g_ref, s_ref, ssq_ref):
    i = pl.program_id(0)
    sb = jnp.dot(ea_ref[...], g_ref[...], preferred_element_type=F32)
    s_ref[...] = sb

    @pl.when(i == 0)
    def _():
        ssq_ref[...] = jnp.zeros_like(ssq_ref)

    ssq_ref[...] += jnp.sum(sb * sb).reshape(1, 1)


def _mlp_body(x_ref, wp_ref, bp_ref, w1_ref, o_ref):
    h = jnp.dot(x_ref[...], wp_ref[...], preferred_element_type=F32)
    h = jnp.maximum(h + bp_ref[...], 0.0)
    o_ref[...] = jnp.dot(h, w1_ref[...], preferred_element_type=F32)


def _ep_body(flag_ref, a_ref, u_ref, v_ref, hw_ref, b_ref, w_ref, bout_ref,
             o_ref):
    asum = a_ref[...]
    h = asum * u_ref[...] + hw_ref[...] * v_ref[...] + b_ref[...]
    f = flag_ref[0, 0]
    h = f * jnp.maximum(h, 0.0) + (1.0 - f) * h
    o_ref[...] = (jnp.dot(h, w_ref[...], preferred_element_type=F32)
                  + bout_ref[...])


def _fast_rsqrt(x):
    i = plsc.bitcast(x, jnp.int32)
    i = jnp.int32(0x5F3759DF) - lax.shift_right_arithmetic(i, 1)
    y = plsc.bitcast(i, F32)
    half = x * 0.5
    for _ in range(4):
        y = y * (1.5 - half * y * y)
    return y


def _make_deg_kernel(NP, EROWS_T):
    npv = NP // NS
    nv = npv // L
    NPH = NP // 2
    HR = EROWS_T // 2
    CAP = HR * 128
    EPAD = NS * EROWS_T * 128
    mesh = plsc.VectorSubcoreMesh(
        core_axis_name="c", subcore_axis_name="s",
        num_cores=1, num_subcores=NS)

    @functools.partial(
        pl.kernel,
        out_type=[jax.ShapeDtypeStruct((NP,), F32)] * 3
        + [jax.ShapeDtypeStruct((2 * EPAD,), jnp.int32),
           jax.ShapeDtypeStruct((2 * EPAD,), jnp.int32),
           jax.ShapeDtypeStruct((2 * EPAD,), F32),
           jax.ShapeDtypeStruct((2 * NS * 2 * L,), jnp.int32)],
        mesh=mesh,
        scratch_types=[
            pltpu.VMEM((HR, 128), jnp.int32),
            pltpu.VMEM((HR, 128), jnp.int32),
            pltpu.VMEM((HR, 128), F32),
            pltpu.VMEM((CAP + L,), jnp.int32),
            pltpu.VMEM((CAP + L,), jnp.int32),
            pltpu.VMEM((CAP + L,), F32),
            pltpu.VMEM((CAP + L,), jnp.int32),
            pltpu.VMEM((CAP + L,), jnp.int32),
            pltpu.VMEM((CAP + L,), F32),
            pltpu.VMEM((L,), jnp.int32),
            pltpu.VMEM((NP,), F32),
            pltpu.VMEM((npv,), F32),
            pltpu.VMEM((npv,), F32),
            pltpu.VMEM((npv,), F32),
            pltpu.VMEM((npv,), F32),
            pltpu.VMEM((L,), F32),
            pltpu.VMEM_SHARED((NS, NP), F32),
        ],
        compiler_params=pltpu.CompilerParams(needs_layout_passes=False),
    )
    def deg_kernel(src_hbm, dst_hbm, s_hbm, c_hbm,
                   dinv_hbm, cdinv_hbm, dinv2_hbm,
                   bsrc_hbm, bdst_hbm, bs_hbm, cnt_hbm,
                   srcv, dstv, sv, b0s, b0d, b0w, b1s, b1d, b1w, cntv,
                   degacc, tmpv, accv, uv, vv, cv, part_sh):
        sid = lax.axis_index("s")

        def zb(i, _):
            degacc[pl.ds(i * L, L)] = jnp.zeros((L,), F32)
            return 0
        lax.fori_loop(0, NP // L, zb, 0)

        pltpu.sync_copy(c_hbm, cv)

        for p in range(2):
            prow = sid * EROWS_T + p * HR
            pltpu.sync_copy(src_hbm.at[pl.ds(prow, HR)], srcv)
            pltpu.sync_copy(dst_hbm.at[pl.ds(prow, HR)], dstv)
            pltpu.sync_copy(s_hbm.at[pl.ds(prow, HR)], sv)

            def scat(k2, offs):
                off0, off1 = offs
                r = k2 // 8
                sl = pl.ds((k2 % 8) * L, L)
                dv = dstv[r, sl]
                wv = sv[r, sl]
                xv = srcv[r, sl]
                plsc.addupdate_scatter(degacc, [dv], wv)

                m0 = dv < NPH
                c0 = jnp.max(plsc.all_reduce_population_count(m0))
                plsc.store_compressed(b0s.at[pl.ds(off0, L)], xv, mask=m0)
                plsc.store_compressed(b0d.at[pl.ds(off0, L)], dv, mask=m0)
                plsc.store_compressed(b0w.at[pl.ds(off0, L)], wv, mask=m0)

                m1 = jnp.logical_not(m0)
                c1 = L - c0
                plsc.store_compressed(b1s.at[pl.ds(off1, L)], xv, mask=m1)
                plsc.store_compressed(b1d.at[pl.ds(off1, L)], dv - NPH, mask=m1)
                plsc.store_compressed(b1w.at[pl.ds(off1, L)], wv, mask=m1)
                return (off0 + c0, off1 + c1)
            off0, off1 = lax.fori_loop(
                0, HR * 8, scat, (jnp.int32(0), jnp.int32(0)))

            for b, (off, bsb, bdb, bwb) in enumerate(
                    [(off0, b0s, b0d, b0w), (off1, b1s, b1d, b1w)]):
                npad_v = (-off) % 256

                def padf(i, _):
                    o = off + i * L
                    spread = (jnp.full((L,), o & 4095, jnp.int32)
                              + lax.iota(jnp.int32, L))
                    bsb[pl.ds(o, L)] = spread
                    bdb[pl.ds(o, L)] = spread
                    bwb[pl.ds(o, L)] = jnp.zeros((L,), F32)
                    return 0
                lax.fori_loop(0, (npad_v + L - 1) // L, padf, 0)

                base = b * EPAD + (sid * EROWS_T + p * HR) * 128
                pltpu.sync_copy(bsb.at[pl.ds(0, CAP)],
                                bsrc_hbm.at[pl.ds(base, CAP)])
                pltpu.sync_copy(bdb.at[pl.ds(0, CAP)],
                                bdst_hbm.at[pl.ds(base, CAP)])
                pltpu.sync_copy(bwb.at[pl.ds(0, CAP)],
                                bs_hbm.at[pl.ds(base, CAP)])

                nch = (off + npad_v) // 256
                cntv[...] = jnp.full((L,), nch, jnp.int32)
                crow = ((b * NS + sid) * 2 + p) * L
                pltpu.sync_copy(cntv, cnt_hbm.at[pl.ds(crow, L)])

        pltpu.sync_copy(degacc, part_sh.at[sid])
        plsc.subcore_barrier()

        base = sid * npv
        pltpu.sync_copy(part_sh.at[0, pl.ds(base, npv)], accv)
        for t in range(1, NS):
            pltpu.sync_copy(part_sh.at[t, pl.ds(base, npv)], tmpv)

            def red(k, _):
                sl = pl.ds(k * L, L)
                accv[sl] += tmpv[sl]
                return 0
            lax.fori_loop(0, nv, red, 0)

        cvec = cv[...]

        def fin(k, _):
            sl = pl.ds(k * L, L)
            deg = 1.0 + cvec * accv[sl]
            y = _fast_rsqrt(deg)
            accv[sl] = y
            uv[sl] = cvec * y
            vv[sl] = y * y
            return 0
        lax.fori_loop(0, nv, fin, 0)

        pltpu.sync_copy(accv, dinv_hbm.at[pl.ds(base, npv)])
        pltpu.sync_copy(uv, cdinv_hbm.at[pl.ds(base, npv)])
        pltpu.sync_copy(vv, dinv2_hbm.at[pl.ds(base, npv)])

    return deg_kernel


def _make_conv_kernel(N, NP, EROWS_T):
    CH = 2
    CE = CH * 128
    NPH = NP // 2
    npv = NPH // NS
    HR = EROWS_T // 2
    CAP = HR * 128
    EPAD = NS * EROWS_T * 128
    mesh = plsc.VectorSubcoreMesh(
        core_axis_name="c", subcore_axis_name="s",
        num_cores=NC, num_subcores=NS)

    @functools.partial(
        pl.kernel,
        out_type=jax.ShapeDtypeStruct((NP, 128), F32),
        mesh=mesh,
        scratch_types=[
            pltpu.VMEM((2 * CE,), jnp.int32),
            pltpu.VMEM((2 * CE,), jnp.int32),
            pltpu.VMEM((2 * CH, 128), jnp.int32),
            pltpu.VMEM((2 * CE,), F32),
            pltpu.VMEM((2 * CE, 128), F32),
            pltpu.VMEM((NP,), F32),
            pltpu.VMEM((L,), F32),
            pltpu.VMEM((L,), jnp.int32),
            pltpu.VMEM((64, 128), F32),
            pltpu.VMEM_SHARED((NPH + 64, 128), F32),
            pltpu.SemaphoreType.DMA,
            pltpu.SemaphoreType.DMA,
        ],
        compiler_params=pltpu.CompilerParams(needs_layout_passes=False),
    )
    def conv_kernel(g_hbm, bsrc_hbm, bdst_hbm, bs_hbm, cnt_hbm, dinv_hbm,
                    out_hbm, srcv, dstv, dstv2, sv, rows, dinvv, wbuf, cntv,
                    zbuf, acc_sh, sem_g, sem_s):
        cid = lax.axis_index("c")
        sid = lax.axis_index("s")

        def zb(i, _):
            zbuf[i // 8, pl.ds((i % 8) * L, L)] = jnp.zeros((L,), F32)
            return 0
        lax.fori_loop(0, 64 * 8, zb, 0)
        for t in range(npv // 64):
            pltpu.sync_copy(zbuf,
                            acc_sh.at[pl.ds(sid * npv + t * 64, 64)])

        pltpu.sync_copy(dinv_hbm, dinvv)
        plsc.subcore_barrier()

        def stage(kk, base_e):
            b = kk & 1
            bo = b * CE
            eb = base_e + kk * CE
            pltpu.sync_copy(bsrc_hbm.at[pl.ds(eb, CE)],
                            srcv.at[pl.ds(bo, CE)])
            pltpu.sync_copy(bdst_hbm.at[pl.ds(eb, CE)],
                            dstv.at[pl.ds(bo, CE)])
            pltpu.sync_copy(bs_hbm.at[pl.ds(eb, CE)], sv.at[pl.ds(bo, CE)])

            def cpd(r, _):
                for j in range(8):
                    dstv2[b * CH + r, pl.ds(j * L, L)] = (
                        dstv[pl.ds(bo + r * 128 + j * L, L)])
                return 0
            lax.fori_loop(0, CH, cpd, 0)

            for j in range(CH):
                pltpu.async_copy(
                    g_hbm.at[srcv.at[pl.ds(bo + j * 128, 128)]],
                    rows.at[pl.ds(bo + j * 128, 128)], sem_g)

        for p in range(2):
            crow = ((cid * NS + sid) * 2 + p) * L
            pltpu.sync_copy(cnt_hbm.at[pl.ds(crow, L)], cntv)
            nch = jnp.max(cntv[...])
            base_e = cid * EPAD + (sid * EROWS_T + p * HR) * 128

            @pl.when(nch > 0)
            def _():
                stage(0, base_e)

            def chunk(kk, _):
                b = kk & 1
                bo = b * CE

                @pl.when(kk >= 1)
                def _():
                    pltpu.make_async_copy(
                        rows.at[pl.ds(0, CE)], acc_sh.at[pl.ds(0, CE)],
                        sem_s).wait()

                pltpu.make_async_copy(
                    g_hbm.at[pl.ds(0, CE)], rows.at[pl.ds(bo, CE)],
                    sem_g).wait()

                @pl.when(kk + 1 < nch)
                def _():
                    stage(kk + 1, base_e)

                def grp(k2, _):
                    sl = pl.ds(bo + k2 * L, L)
                    svec = sv[sl]
                    srcvec = srcv[sl]
                    dg = plsc.load_gather(dinvv, [srcvec])
                    wv = svec * dg
                    r0 = bo + k2 * L
                    for r in range(L):
                        wb = lax.gather(
                            wv, jnp.full((L, 1), r, jnp.int32),
                            lax.GatherDimensionNumbers(
                                offset_dims=(), collapsed_slice_dims=(0,),
                                start_index_map=(0,)),
                            (1,),
                            mode=lax.GatherScatterMode.PROMISE_IN_BOUNDS)
                        for jj in range(8):
                            rows[r0 + r, pl.ds(jj * L, L)] *= wb
                    return 0
                lax.fori_loop(0, CE // L, grp, 0)

                for j in range(CH):
                    pltpu.async_copy(rows.at[pl.ds(bo + j * 128, 128)],
                                     acc_sh.at[dstv2.at[b * CH + j]], sem_s,
                                     add=True)
                return 0
            lax.fori_loop(0, nch, chunk, 0)

            @pl.when(nch >= 1)
            def _():
                pltpu.make_async_copy(
                    rows.at[pl.ds(0, CE)], acc_sh.at[pl.ds(0, CE)],
                    sem_s).wait()

        plsc.subcore_barrier()
        pltpu.sync_copy(acc_sh.at[pl.ds(sid * npv, npv)],
                        out_hbm.at[pl.ds(cid * NPH + sid * npv, npv)])

    return conv_kernel


def kernel(x, edge_index, edge_attr, Wp, bp, W1, b1, W2, b2, Wo, bo):
    N, D = x.shape
    E, DE = edge_attr.shape
    H = W1.shape[1]
    assert D == 128 and H == 128 and DE == 16

    NP = -(-N // 2048) * 2048
    EROWS_T = -(-E // (NS * 128 * 4)) * 4
    EROWS = NS * EROWS_T
    EPAD = EROWS * 128

    src = edge_index[0]
    dst = edge_index[1]

    ER = E // 8
    ea2 = edge_attr.reshape(ER, 128)
    gmat = jnp.repeat(jnp.eye(8, dtype=F32), 16, axis=0)
    BE = 4000
    s8, ssq = pl.pallas_call(
        _edge_sum_body,
        grid=(ER // BE,),
        in_specs=[
            pl.BlockSpec((BE, 128), lambda i: (i, 0)),
            pl.BlockSpec((128, 8), lambda i: (0, 0)),
        ],
        out_specs=[
            pl.BlockSpec((BE, 8), lambda i: (i, 0)),
            pl.BlockSpec((1, 1), lambda i: (0, 0)),
        ],
        out_shape=[
            jax.ShapeDtypeStruct((ER, 8), F32),
            jax.ShapeDtypeStruct((1, 1), F32),
        ],
    )(ea2, gmat)

    c = 1.0 / jnp.maximum(jnp.sqrt(ssq[0, 0]), 1e-12)

    BN = 2000
    mlp_call = pl.pallas_call(
        _mlp_body,
        grid=(N // BN,),
        in_specs=[
            pl.BlockSpec((BN, D), lambda i: (i, 0)),
            pl.BlockSpec((D, H), lambda i: (0, 0)),
            pl.BlockSpec((1, H), lambda i: (0, 0)),
            pl.BlockSpec((H, H), lambda i: (0, 0)),
        ],
        out_specs=pl.BlockSpec((BN, H), lambda i: (i, 0)),
        out_shape=jax.ShapeDtypeStruct((N, H), F32),
    )
    hw1 = mlp_call(x, Wp, bp.reshape(1, H), W1)

    pad = EPAD - E
    padi = jnp.arange(pad, dtype=jnp.int32) % N
    src_p = jnp.concatenate([src, padi]).reshape(EROWS, 128)
    dst_p = jnp.concatenate([dst, padi]).reshape(EROWS, 128)
    s_p = jnp.concatenate([s8.reshape(E),
                           jnp.zeros((pad,), F32)]).reshape(EROWS, 128)
    cq = jnp.full((L,), c, F32)

    deg_kernel = _make_deg_kernel(NP, EROWS_T)
    dinv_f, cdinv_f, dinv2_f, bsrc, bdst, bs, cnts = deg_kernel(
        src_p, dst_p, s_p, cq)
    u_col = cdinv_f[:N].reshape(N, 1)
    v_col = dinv2_f[:N].reshape(N, 1)

    conv_kernel = _make_conv_kernel(N, NP, EROWS_T)

    ep_call = pl.pallas_call(
        _ep_body,
        grid=(N // BN,),
        in_specs=[
            pl.BlockSpec(memory_space=pltpu.SMEM),
            pl.BlockSpec((BN, 128), lambda i: (i, 0)),
            pl.BlockSpec((BN, 1), lambda i: (i, 0)),
            pl.BlockSpec((BN, 1), lambda i: (i, 0)),
            pl.BlockSpec((BN, H), lambda i: (i, 0)),
            pl.BlockSpec((1, H), lambda i: (0, 0)),
            pl.BlockSpec((H, H), lambda i: (0, 0)),
            pl.BlockSpec((1, H), lambda i: (0, 0)),
        ],
        out_specs=pl.BlockSpec((BN, H), lambda i: (i, 0)),
        out_shape=jax.ShapeDtypeStruct((N, H), F32),
    )

    def step(hw, ws):
        w_s, b_s, bout_s, flag_s = ws
        acc = conv_kernel(hw, bsrc, bdst, bs, cnts, dinv_f)
        hw_next = ep_call(flag_s, acc, u_col, v_col, hw, b_s, w_s, bout_s)
        return hw_next, 0.0

    ws = (
        jnp.stack([W2, Wo]),
        jnp.stack([b1.reshape(1, H), b2.reshape(1, H)]),
        jnp.stack([jnp.zeros((1, H), F32), bo.reshape(1, H)]),
        jnp.stack([jnp.ones((1, 1), F32), jnp.zeros((1, 1), F32)]),
    )
    out, _ = lax.scan(step, hw1, ws)
    return out
